# speedup vs baseline: 1.3773x; 1.3773x over previous
import functools

import jax
import jax.numpy as jnp
from jax import lax
from jax.experimental import pallas as pl
from jax.experimental.pallas import tpu as pltpu
from jax.experimental.pallas import tpu_sc as plsc

N_NODES = 50000
C_IN = 128
C_OUT = 128
K = 9
NW = 32
B = 64
NCHUNK = 25
NPAD = NW * B * NCHUNK
MM_BLK = 512


def _mm_body(x_ref, w_ref, y_ref):
    y_ref[0] = jnp.dot(x_ref[...], w_ref[0], preferred_element_type=jnp.float32)


def _tc_matmul(fpad, wr):
    return pl.pallas_call(
        _mm_body,
        grid=(NPAD // MM_BLK, K),
        in_specs=[
            pl.BlockSpec((MM_BLK, C_IN), lambda i, k: (i, 0)),
            pl.BlockSpec((1, C_IN, C_OUT), lambda i, k: (k, 0, 0)),
        ],
        out_specs=pl.BlockSpec((1, MM_BLK, C_OUT), lambda i, k: (k, i, 0)),
        out_shape=jax.ShapeDtypeStruct((K, NPAD, C_OUT), jnp.float32),
    )(fpad, wr)


def _sc_body(y_hbm, idx_hbm, b_hbm, out_hbm, b_v, idx_v, rows_v, out_v, sem):
    wid = lax.axis_index("s") * 2 + lax.axis_index("c")
    pltpu.sync_copy(b_hbm, b_v)
    rows_per_w = NPAD // NW

    def chunk(c, carry):
        base = pl.multiple_of(wid * rows_per_w + c * B, 64)
        for k in range(K):
            pltpu.sync_copy(idx_hbm.at[pl.ds(k * NPAD + base, B)], idx_v.at[k])
        for k in range(K):
            for t in range(B // 16):
                s = pl.ds(t * 16, 16)
                v = idx_v[k, s]
                idx_v[k, s] = jnp.where(v < 0, N_NODES, v) + k * NPAD
        copies = [
            pltpu.async_copy(y_hbm.at[idx_v.at[k]], rows_v.at[k], sem)
            for k in range(K)
        ]
        for cp in copies:
            cp.wait()

        def acc(i, carry2):
            for cseg in range(C_OUT // 16):
                s = pl.ds(cseg * 16, 16)
                a = b_v[s]
                for k in range(K):
                    a = a + rows_v[k, i, s]
                out_v[i, s] = a
            return carry2

        lax.fori_loop(0, B, acc, 0)
        pltpu.sync_copy(out_v, out_hbm.at[pl.ds(base, B)])
        return carry

    lax.fori_loop(0, NCHUNK, chunk, 0)


def _sc_gather_sum(yflat, idx_flat, b):
    mesh = plsc.VectorSubcoreMesh(core_axis_name="c", subcore_axis_name="s")
    fn = pl.kernel(
        _sc_body,
        mesh=mesh,
        out_type=jax.ShapeDtypeStruct((NPAD, C_OUT), jnp.float32),
        scratch_types=[
            pltpu.VMEM((C_OUT,), jnp.float32),
            pltpu.VMEM((K, B), jnp.int32),
            pltpu.VMEM((K, B, C_OUT), jnp.float32),
            pltpu.VMEM((B, C_OUT), jnp.float32),
            pltpu.SemaphoreType.DMA,
        ],
    )
    return fn(yflat, idx_flat, b)


def kernel(features, neigh_idx, W, b):
    N, C = features.shape
    fpad = jnp.zeros((NPAD, C), features.dtype).at[:N].set(features)
    wr = W.reshape(C_OUT, K, C_IN).transpose(1, 2, 0)
    y = _tc_matmul(fpad, wr)
    yflat = y.reshape(K * NPAD, C_OUT)
    idx_pad = jnp.full((NPAD, K), -1, neigh_idx.dtype).at[:N].set(neigh_idx)
    idx_flat = idx_pad.T.reshape(-1).astype(jnp.int32)
    out = _sc_gather_sum(yflat, idx_flat, b)
    return out[:N]


# speedup vs baseline: 1.5571x; 1.1305x over previous
import functools

import jax
import jax.numpy as jnp
from jax import lax
from jax.experimental import pallas as pl
from jax.experimental.pallas import tpu as pltpu
from jax.experimental.pallas import tpu_sc as plsc

N_NODES = 50000
C_IN = 128
C_OUT = 128
K = 9
NW = 32
B = 48
NCHUNK = 33
NPAD = NW * B * NCHUNK
ROWS_PER_W = NPAD // NW
MM_BLK = 512


def _mm_body(x_ref, w_ref, y_ref):
    y_ref[0] = jnp.dot(x_ref[...], w_ref[0], preferred_element_type=jnp.float32)


def _tc_matmul(fpad, wr):
    return pl.pallas_call(
        _mm_body,
        grid=(NPAD // MM_BLK, K),
        in_specs=[
            pl.BlockSpec((MM_BLK, C_IN), lambda i, k: (i, 0)),
            pl.BlockSpec((1, C_IN, C_OUT), lambda i, k: (k, 0, 0)),
        ],
        out_specs=pl.BlockSpec((1, MM_BLK, C_OUT), lambda i, k: (k, i, 0)),
        out_shape=jax.ShapeDtypeStruct((K, NPAD, C_OUT), jnp.float32),
    )(fpad, wr)


def _remap_body(idx_ref, gidx_ref):
    v = idx_ref[...]
    koff = lax.broadcasted_iota(jnp.int32, (K, NPAD), 0) * NPAD
    gidx_ref[...] = jnp.where(v < 0, N_NODES, v) + koff


def _tc_remap(idx_t):
    return pl.pallas_call(
        _remap_body,
        out_shape=jax.ShapeDtypeStruct((K, NPAD), jnp.int32),
    )(idx_t)


def _sc_body(y_hbm, gidx_hbm, b_hbm, out_hbm,
             b_v, gidx_v, rows0, out0, gsem0):
    wid = lax.axis_index("s") * 2 + lax.axis_index("c")
    pltpu.sync_copy(b_hbm, b_v)
    pltpu.sync_copy(gidx_hbm.at[wid], gidx_v)
    wbase = wid * ROWS_PER_W

    def chunk(c, carry):
        copies = [pltpu.async_copy(y_hbm.at[gidx_v.at[k, c]], rows0.at[k],
                                   gsem0) for k in range(K)]
        for cp in copies:
            cp.wait()

        def acc(i, carry2):
            for cseg in range(C_OUT // 16):
                s = pl.ds(cseg * 16, 16)
                a = b_v[s]
                for k in range(K):
                    a = a + rows0[k, i, s]
                out0[i, s] = a
            return carry2

        lax.fori_loop(0, B, acc, 0)
        pltpu.sync_copy(out0, out_hbm.at[pl.ds(wbase + c * B, B)])
        return carry

    lax.fori_loop(0, NCHUNK, chunk, 0)


def _sc_gather_sum(yflat, gidx4, b):
    mesh = plsc.VectorSubcoreMesh(core_axis_name="c", subcore_axis_name="s")
    fn = pl.kernel(
        _sc_body,
        mesh=mesh,
        out_type=jax.ShapeDtypeStruct((NPAD, C_OUT), jnp.float32),
        scratch_types=[
            pltpu.VMEM((C_OUT,), jnp.float32),
            pltpu.VMEM((K, NCHUNK, B), jnp.int32),
            pltpu.VMEM((K, B, C_OUT), jnp.float32),
            pltpu.VMEM((B, C_OUT), jnp.float32),
            pltpu.SemaphoreType.DMA,
        ],
    )
    return fn(yflat, gidx4, b)


def kernel(features, neigh_idx, W, b):
    N, C = features.shape
    fpad = jnp.zeros((NPAD, C), features.dtype).at[:N].set(features)
    wr = W.reshape(C_OUT, K, C_IN).transpose(1, 2, 0)
    y = _tc_matmul(fpad, wr)
    yflat = y.reshape(K * NPAD, C_OUT)
    idx_pad = jnp.full((NPAD, K), -1, neigh_idx.dtype).at[:N].set(neigh_idx)
    idx_t = idx_pad.T.astype(jnp.int32)
    gidx = _tc_remap(idx_t)
    gidx4 = gidx.reshape(K, NW, NCHUNK, B).transpose(1, 0, 2, 3)
    out = _sc_gather_sum(yflat, gidx4, b)
    return out[:N]
